# baseline (speedup 1.0000x reference)
import functools

import jax
import jax.numpy as jnp
from jax import lax
from jax.experimental import pallas as pl
from jax.experimental.pallas import tpu as pltpu
from jax.experimental.pallas import tpu_sc as plsc

NC = 2
NS = 16
NW = NC * NS
LANES = 16
K = 128
G = 40


def _sc_agg_call(x, src3, dst3, with_deg):
    Npad, D = x.shape
    NG = src3.shape[1]
    rows_per_tile = Npad // NS
    mesh = plsc.VectorSubcoreMesh(core_axis_name="c", subcore_axis_name="s")

    out_type = [jax.ShapeDtypeStruct((NC, Npad, D), jnp.float32)]
    scratch = [
        pltpu.VMEM_SHARED((Npad, D), jnp.float32),
        pltpu.VMEM((G, K), jnp.int32),
        pltpu.VMEM((G, K), jnp.int32),
        pltpu.VMEM((K, D), jnp.float32),
        pltpu.VMEM((K, D), jnp.float32),
        pltpu.SemaphoreType.DMA,
        pltpu.SemaphoreType.DMA,
    ]
    if with_deg:
        out_type.append(jax.ShapeDtypeStruct((NC * Npad,), jnp.float32))
        scratch += [
            pltpu.VMEM_SHARED((Npad,), jnp.float32),
            pltpu.VMEM((K,), jnp.float32),
            pltpu.VMEM((rows_per_tile,), jnp.float32),
        ]

    def body(x_hbm, src_hbm, dst_hbm, *refs):
        if with_deg:
            (out_hbm, dout_hbm, acc, src_l, dst_l, rows_a, rows_b,
             sem_a, sem_b, dacc, ones, dbuf) = refs
        else:
            (out_hbm, acc, src_l, dst_l, rows_a, rows_b,
             sem_a, sem_b) = refs
        cid = lax.axis_index("c")
        sid = lax.axis_index("s")
        wid = cid * NS + sid
        base = sid * rows_per_tile

        def zfill(i, c):
            for j in range(D // LANES):
                rows_a[i, pl.ds(j * LANES, LANES)] = jnp.zeros((LANES,), jnp.float32)
            return c
        lax.fori_loop(0, K, zfill, 0)

        if with_deg:
            for j in range(K // LANES):
                ones[pl.ds(j * LANES, LANES)] = jnp.zeros((LANES,), jnp.float32)

        def zacc(i, c):
            pltpu.sync_copy(rows_a, acc.at[pl.ds(base + i * K, K)])
            if with_deg:
                pltpu.sync_copy(ones, dacc.at[pl.ds(base + i * K, K)])
            return c
        lax.fori_loop(0, rows_per_tile // K, zacc, 0)

        if with_deg:
            for j in range(K // LANES):
                ones[pl.ds(j * LANES, LANES)] = jnp.ones((LANES,), jnp.float32)

        plsc.subcore_barrier()

        def scat(buf, j):
            pltpu.sync_copy(buf, acc.at[dst_l.at[j]], add=True)
            if with_deg:
                pltpu.sync_copy(ones, dacc.at[dst_l.at[j]], add=True)

        def group(g, c):
            pltpu.sync_copy(src_hbm.at[wid, g], src_l)
            pltpu.sync_copy(dst_hbm.at[wid, g], dst_l)
            pltpu.async_copy(x_hbm.at[src_l.at[0]], rows_a, sem_a)

            def pair(p, c2):
                j0 = 2 * p
                j1 = j0 + 1
                pltpu.make_async_copy(
                    x_hbm.at[src_l.at[j0]], rows_a, sem_a).wait()
                pltpu.async_copy(x_hbm.at[src_l.at[j1]], rows_b, sem_b)
                scat(rows_a, j0)
                pltpu.make_async_copy(
                    x_hbm.at[src_l.at[j1]], rows_b, sem_b).wait()

                @pl.when(p < G // 2 - 1)
                def _():
                    pltpu.async_copy(x_hbm.at[src_l.at[j0 + 2]], rows_a, sem_a)

                scat(rows_b, j1)
                return c2
            lax.fori_loop(0, G // 2, pair, 0)
            return c
        lax.fori_loop(0, NG, group, 0)

        plsc.subcore_barrier()

        def out_copy(j, c):
            r = base + j * K
            pltpu.sync_copy(acc.at[pl.ds(r, K)], out_hbm.at[cid, pl.ds(r, K)])
            return c
        lax.fori_loop(0, rows_per_tile // K, out_copy, 0)
        if with_deg:
            pltpu.sync_copy(dacc.at[pl.ds(base, rows_per_tile)], dbuf)
            pltpu.sync_copy(dbuf, dout_hbm.at[pl.ds(cid * Npad + base,
                                                    rows_per_tile)])

    f = pl.kernel(body, out_type=out_type, mesh=mesh, scratch_types=scratch)
    return f(x, src3, dst3)


def _deg_inv_col(d0, d1):
    deg = d0[:, 0:1] + d1[:, 0:1]
    return jnp.where(deg > 0.0, 1.0 / jnp.maximum(deg, 1.0), 0.0)


def _layer_body(x_ref, a0_ref, a1_ref, d0_ref, d1_ref, w_ref, b_ref, o_ref,
                *, relu):
    dinv = _deg_inv_col(d0_ref[...], d1_ref[...])
    h = x_ref[...] + (a0_ref[...] + a1_ref[...]) * dinv
    y = jnp.dot(h, w_ref[...], preferred_element_type=jnp.float32) + b_ref[...]
    o_ref[...] = jnp.maximum(y, 0.0) if relu else y


def _tc_layer(x, a0, a1, d0, d1, w, b, relu):
    Npad, D = x.shape
    H = w.shape[1]
    Nb = 512
    grid = Npad // Nb
    return pl.pallas_call(
        functools.partial(_layer_body, relu=relu),
        grid=(grid,),
        in_specs=[
            pl.BlockSpec((Nb, D), lambda i: (i, 0)),
            pl.BlockSpec((Nb, D), lambda i: (i, 0)),
            pl.BlockSpec((Nb, D), lambda i: (i, 0)),
            pl.BlockSpec((Nb, LANES), lambda i: (i, 0)),
            pl.BlockSpec((Nb, LANES), lambda i: (i, 0)),
            pl.BlockSpec((D, H), lambda i: (0, 0)),
            pl.BlockSpec((1, H), lambda i: (0, 0)),
        ],
        out_specs=pl.BlockSpec((Nb, H), lambda i: (i, 0)),
        out_shape=jax.ShapeDtypeStruct((Npad, H), jnp.float32),
    )(x, a0, a1, d0, d1, w, b)


def _final_body(x_ref, a0_ref, a1_ref, d0_ref, d1_ref, w_ref, b_ref,
                wc_ref, bc_ref, o_ref, m_ref, *, n_valid, nb, grid):
    i = pl.program_id(0)
    dinv = _deg_inv_col(d0_ref[...], d1_ref[...])
    h = x_ref[...] + (a0_ref[...] + a1_ref[...]) * dinv
    y = jnp.dot(h, w_ref[...], preferred_element_type=jnp.float32) + b_ref[...]
    rowid = i * nb + lax.broadcasted_iota(jnp.int32, y.shape, 0)
    y = jnp.where(rowid < n_valid, y, -1e30)
    bm = jnp.max(y, axis=0, keepdims=True)

    @pl.when(i == 0)
    def _():
        m_ref[0:1] = bm

    @pl.when(i > 0)
    def _():
        m_ref[0:1] = jnp.maximum(m_ref[0:1], bm)

    @pl.when(i == grid - 1)
    def _():
        hg = m_ref[0:1]
        o_ref[...] = (jnp.dot(hg, wc_ref[...], preferred_element_type=jnp.float32)
                      + bc_ref[...])


def _tc_final(x, a0, a1, d0, d1, w, b, wc, bc, n_valid):
    Npad, D = x.shape
    H = w.shape[1]
    C = wc.shape[1]
    Nb = 512
    grid = Npad // Nb
    return pl.pallas_call(
        functools.partial(_final_body, n_valid=n_valid, nb=Nb, grid=grid),
        grid=(grid,),
        in_specs=[
            pl.BlockSpec((Nb, D), lambda i: (i, 0)),
            pl.BlockSpec((Nb, D), lambda i: (i, 0)),
            pl.BlockSpec((Nb, D), lambda i: (i, 0)),
            pl.BlockSpec((Nb, LANES), lambda i: (i, 0)),
            pl.BlockSpec((Nb, LANES), lambda i: (i, 0)),
            pl.BlockSpec((D, H), lambda i: (0, 0)),
            pl.BlockSpec((1, H), lambda i: (0, 0)),
            pl.BlockSpec((H, C), lambda i: (0, 0)),
            pl.BlockSpec((1, C), lambda i: (0, 0)),
        ],
        out_specs=pl.BlockSpec((1, C), lambda i: (0, 0)),
        out_shape=jax.ShapeDtypeStruct((1, C), jnp.float32),
        scratch_shapes=[pltpu.VMEM((8, 128), jnp.float32)],
    )(x, a0, a1, d0, d1, w, b, wc, bc)


def kernel(in_feat, edge_index, W1, b1, W2, b2, Wc, bc):
    N, D = in_feat.shape
    E = edge_index.shape[1]

    Npad = ((N + 10239) // 10240) * 10240
    x = jnp.pad(in_feat, ((0, Npad - N), (0, 0)))

    Echunk = NW * K * G
    Epad = ((E + Echunk - 1) // Echunk) * Echunk
    src = edge_index[0]
    dst = edge_index[1]
    if Epad != E:
        fill = jnp.full((Epad - E,), N, dtype=jnp.int32)
        src = jnp.concatenate([src, fill])
        dst = jnp.concatenate([dst, fill])
    NG = Epad // (NW * G * K)
    src3 = src.reshape(NW, NG, G, K)
    dst3 = dst.reshape(NW, NG, G, K)

    b1r = b1.reshape(1, -1)
    b2r = b2.reshape(1, -1)
    bcr = bc.reshape(1, -1)

    agg1, degp = _sc_agg_call(x, src3, dst3, with_deg=True)
    degp = degp.reshape(NC, Npad)
    d0 = jnp.broadcast_to(degp[0][:, None], (Npad, LANES))
    d1 = jnp.broadcast_to(degp[1][:, None], (Npad, LANES))
    h1 = _tc_layer(x, agg1[0], agg1[1], d0, d1, W1, b1r, relu=True)

    (agg2,) = _sc_agg_call(h1, src3, dst3, with_deg=False)
    out = _tc_final(h1, agg2[0], agg2[1], d0, d1, W2, b2r, Wc, bcr, N)
    return out


# speedup vs baseline: 1.8997x; 1.8997x over previous
import functools

import jax
import jax.numpy as jnp
from jax import lax
from jax.experimental import pallas as pl
from jax.experimental.pallas import tpu as pltpu
from jax.experimental.pallas import tpu_sc as plsc

NC = 2
NS = 16
NW = NC * NS
LANES = 16
K = 80
G = 25


def _sc_agg_call(x, src3, dst3, with_deg):
    Npad, D = x.shape
    NG = src3.shape[1]
    rows_per_tile = Npad // NS
    mesh = plsc.VectorSubcoreMesh(core_axis_name="c", subcore_axis_name="s")

    out_type = [jax.ShapeDtypeStruct((NC, Npad, D), jnp.float32)]
    scratch = [
        pltpu.VMEM_SHARED((Npad, D), jnp.float32),
        pltpu.VMEM((G, K), jnp.int32),
        pltpu.VMEM((G, K), jnp.int32),
        pltpu.VMEM((K, D), jnp.float32),
        pltpu.SemaphoreType.DMA,
    ]
    if with_deg:
        out_type.append(jax.ShapeDtypeStruct((NC * Npad,), jnp.float32))
        scratch += [
            pltpu.VMEM_SHARED((Npad,), jnp.float32),
            pltpu.VMEM((K,), jnp.float32),
            pltpu.VMEM((rows_per_tile,), jnp.float32),
            pltpu.SemaphoreType.DMA,
        ]

    def body(x_hbm, src_hbm, dst_hbm, *refs):
        if with_deg:
            (out_hbm, dout_hbm, acc, src_l, dst_l, rows_a,
             sem_a, dacc, ones, dbuf, sem_d) = refs
        else:
            (out_hbm, acc, src_l, dst_l, rows_a, sem_a) = refs
        cid = lax.axis_index("c")
        sid = lax.axis_index("s")
        wid = cid * NS + sid
        base = sid * rows_per_tile

        def zfill(i, c):
            for j in range(D // LANES):
                rows_a[i, pl.ds(j * LANES, LANES)] = jnp.zeros((LANES,), jnp.float32)
            return c
        lax.fori_loop(0, K, zfill, 0)

        if with_deg:
            for j in range(K // LANES):
                ones[pl.ds(j * LANES, LANES)] = jnp.zeros((LANES,), jnp.float32)

        def zacc(i, c):
            pltpu.sync_copy(rows_a, acc.at[pl.ds(base + i * K, K)])
            if with_deg:
                pltpu.sync_copy(ones, dacc.at[pl.ds(base + i * K, K)])
            return c
        lax.fori_loop(0, rows_per_tile // K, zacc, 0)

        if with_deg:
            for j in range(K // LANES):
                ones[pl.ds(j * LANES, LANES)] = jnp.ones((LANES,), jnp.float32)

        plsc.subcore_barrier()

        def scat(buf, j):
            pltpu.sync_copy(buf, acc.at[dst_l.at[j]], add=True)
            if with_deg:
                pltpu.async_copy(ones, dacc.at[dst_l.at[j]], sem_d, add=True)

        def group(g, c):
            pltpu.sync_copy(src_hbm.at[wid, g], src_l)
            pltpu.sync_copy(dst_hbm.at[wid, g], dst_l)

            def step(j, c2):
                pltpu.async_copy(x_hbm.at[src_l.at[j]], rows_a, sem_a).wait()
                scat(rows_a, j)
                return c2
            lax.fori_loop(0, G, step, 0)

            if with_deg:
                def drain(j, c2):
                    pltpu.make_async_copy(
                        ones, dacc.at[dst_l.at[j]], sem_d).wait()
                    return c2
                lax.fori_loop(0, G, drain, 0)
            return c
        lax.fori_loop(0, NG, group, 0)

        plsc.subcore_barrier()

        def out_copy(j, c):
            r = base + j * K
            pltpu.sync_copy(acc.at[pl.ds(r, K)], out_hbm.at[cid, pl.ds(r, K)])
            return c
        lax.fori_loop(0, rows_per_tile // K, out_copy, 0)
        if with_deg:
            pltpu.sync_copy(dacc.at[pl.ds(base, rows_per_tile)], dbuf)
            pltpu.sync_copy(dbuf, dout_hbm.at[pl.ds(cid * Npad + base,
                                                    rows_per_tile)])

    f = pl.kernel(body, out_type=out_type, mesh=mesh, scratch_types=scratch)
    return f(x, src3, dst3)


def _deg_inv_col(d0, d1):
    deg = d0[:, 0:1] + d1[:, 0:1]
    return jnp.where(deg > 0.0, 1.0 / jnp.maximum(deg, 1.0), 0.0)


def _layer_body(x_ref, a0_ref, a1_ref, d0_ref, d1_ref, w_ref, b_ref, o_ref,
                *, relu):
    dinv = _deg_inv_col(d0_ref[...], d1_ref[...])
    h = x_ref[...] + (a0_ref[...] + a1_ref[...]) * dinv
    y = jnp.dot(h, w_ref[...], preferred_element_type=jnp.float32) + b_ref[...]
    o_ref[...] = jnp.maximum(y, 0.0) if relu else y


def _tc_layer(x, a0, a1, d0, d1, w, b, relu):
    Npad, D = x.shape
    H = w.shape[1]
    Nb = 512
    grid = Npad // Nb
    return pl.pallas_call(
        functools.partial(_layer_body, relu=relu),
        grid=(grid,),
        in_specs=[
            pl.BlockSpec((Nb, D), lambda i: (i, 0)),
            pl.BlockSpec((Nb, D), lambda i: (i, 0)),
            pl.BlockSpec((Nb, D), lambda i: (i, 0)),
            pl.BlockSpec((Nb, LANES), lambda i: (i, 0)),
            pl.BlockSpec((Nb, LANES), lambda i: (i, 0)),
            pl.BlockSpec((D, H), lambda i: (0, 0)),
            pl.BlockSpec((1, H), lambda i: (0, 0)),
        ],
        out_specs=pl.BlockSpec((Nb, H), lambda i: (i, 0)),
        out_shape=jax.ShapeDtypeStruct((Npad, H), jnp.float32),
    )(x, a0, a1, d0, d1, w, b)


def _final_body(x_ref, a0_ref, a1_ref, d0_ref, d1_ref, w_ref, b_ref,
                wc_ref, bc_ref, o_ref, m_ref, *, n_valid, nb, grid):
    i = pl.program_id(0)
    dinv = _deg_inv_col(d0_ref[...], d1_ref[...])
    h = x_ref[...] + (a0_ref[...] + a1_ref[...]) * dinv
    y = jnp.dot(h, w_ref[...], preferred_element_type=jnp.float32) + b_ref[...]
    rowid = i * nb + lax.broadcasted_iota(jnp.int32, y.shape, 0)
    y = jnp.where(rowid < n_valid, y, -1e30)
    bm = jnp.max(y, axis=0, keepdims=True)

    @pl.when(i == 0)
    def _():
        m_ref[0:1] = bm

    @pl.when(i > 0)
    def _():
        m_ref[0:1] = jnp.maximum(m_ref[0:1], bm)

    @pl.when(i == grid - 1)
    def _():
        hg = m_ref[0:1]
        o_ref[...] = (jnp.dot(hg, wc_ref[...], preferred_element_type=jnp.float32)
                      + bc_ref[...])


def _tc_final(x, a0, a1, d0, d1, w, b, wc, bc, n_valid):
    Npad, D = x.shape
    H = w.shape[1]
    C = wc.shape[1]
    Nb = 512
    grid = Npad // Nb
    return pl.pallas_call(
        functools.partial(_final_body, n_valid=n_valid, nb=Nb, grid=grid),
        grid=(grid,),
        in_specs=[
            pl.BlockSpec((Nb, D), lambda i: (i, 0)),
            pl.BlockSpec((Nb, D), lambda i: (i, 0)),
            pl.BlockSpec((Nb, D), lambda i: (i, 0)),
            pl.BlockSpec((Nb, LANES), lambda i: (i, 0)),
            pl.BlockSpec((Nb, LANES), lambda i: (i, 0)),
            pl.BlockSpec((D, H), lambda i: (0, 0)),
            pl.BlockSpec((1, H), lambda i: (0, 0)),
            pl.BlockSpec((H, C), lambda i: (0, 0)),
            pl.BlockSpec((1, C), lambda i: (0, 0)),
        ],
        out_specs=pl.BlockSpec((1, C), lambda i: (0, 0)),
        out_shape=jax.ShapeDtypeStruct((1, C), jnp.float32),
        scratch_shapes=[pltpu.VMEM((8, 128), jnp.float32)],
    )(x, a0, a1, d0, d1, w, b, wc, bc)


def kernel(in_feat, edge_index, W1, b1, W2, b2, Wc, bc):
    N, D = in_feat.shape
    E = edge_index.shape[1]

    Npad = ((N + 10239) // 10240) * 10240
    x = jnp.pad(in_feat, ((0, Npad - N), (0, 0)))

    Echunk = NW * K * G
    Epad = ((E + Echunk - 1) // Echunk) * Echunk
    src = edge_index[0]
    dst = edge_index[1]
    if Epad != E:
        fill = jnp.full((Epad - E,), N, dtype=jnp.int32)
        src = jnp.concatenate([src, fill])
        dst = jnp.concatenate([dst, fill])
    NG = Epad // (NW * G * K)
    src3 = src.reshape(NW, NG, G, K)
    dst3 = dst.reshape(NW, NG, G, K)

    b1r = b1.reshape(1, -1)
    b2r = b2.reshape(1, -1)
    bcr = bc.reshape(1, -1)

    agg1, degp = _sc_agg_call(x, src3, dst3, with_deg=True)
    degp = degp.reshape(NC, Npad)
    d0 = jnp.broadcast_to(degp[0][:, None], (Npad, LANES))
    d1 = jnp.broadcast_to(degp[1][:, None], (Npad, LANES))
    h1 = _tc_layer(x, agg1[0], agg1[1], d0, d1, W1, b1r, relu=True)

    (agg2,) = _sc_agg_call(h1, src3, dst3, with_deg=False)
    out = _tc_final(h1, agg2[0], agg2[1], d0, d1, W2, b2r, Wc, bcr, N)
    return out
